# baseline (speedup 1.0000x reference)
import functools

import jax
import jax.numpy as jnp
from jax import lax
from jax.experimental import pallas as pl
from jax.experimental.pallas import tpu as pltpu
from jax.experimental.pallas import tpu_sc as plsc

N_PROPS = 50000
N_ACTS = 100000
D = 256
DW = D // 2
NC, NS = 2, 16
NW = NC * NS
C = 64
CH0 = 38
CH1 = 62
SLAB = (CH0 + CH1) * C
MAXCH = max(CH0, CH1)
PAD_ACTS = NS * SLAB

FULL_MAX = N_ACTS - C
STRAD_OFF = (N_ACTS // C) * C
STRAD_ROWS = N_ACTS - STRAD_OFF

MM_ROWS = 2000


def _mm_body(x_ref, w_ref, b_ref, o0_ref, o1_ref, o2_ref):
    x = x_ref[...]
    for k, o_ref in enumerate((o0_ref, o1_ref, o2_ref)):
        p = jnp.dot(x, w_ref[k * D:(k + 1) * D, :],
                    preferred_element_type=jnp.float32)
        if k == 0:
            p = p + b_ref[...]
        bits = lax.bitcast_convert_type(p, jnp.uint32)
        r16 = (bits + 0x7FFF + ((bits >> 16) & 1)) >> 16
        w = r16[:, :DW] | (r16[:, DW:] << 16)
        o_ref[...] = lax.bitcast_convert_type(w, jnp.int32)


def _project_tables(prop2d, W, b):
    return pl.pallas_call(
        _mm_body,
        grid=(N_PROPS // MM_ROWS,),
        in_specs=[
            pl.BlockSpec((MM_ROWS, D), lambda i: (i, 0)),
            pl.BlockSpec((3 * D, D), lambda i: (0, 0)),
            pl.BlockSpec((1, D), lambda i: (0, 0)),
        ],
        out_specs=[pl.BlockSpec((MM_ROWS, DW), lambda i: (i, 0))] * 3,
        out_shape=[jax.ShapeDtypeStruct((N_PROPS, DW), jnp.int32)] * 3,
    )(prop2d, W, b.reshape(1, D))


def _sc_body(p0_hbm, p1_hbm, p2_hbm, i0_hbm, i1_hbm, i2_hbm, out_hbm,
             ia0, ia1, ia2, r0a, r1a, r2a, r0b, r1b, r2b, oa, ob,
             sem_a, sem_b, osem_a, osem_b):
    cid = lax.axis_index("c")
    sid = lax.axis_index("s")
    my_chunks = jnp.where(cid == 0, CH0, CH1)
    base = sid * SLAB + cid * (CH0 * C)

    @pl.when(cid == 0)
    def _():
        pltpu.sync_copy(i0_hbm.at[pl.ds(base, CH0 * C)], ia0.at[pl.ds(0, CH0 * C)])
        pltpu.sync_copy(i1_hbm.at[pl.ds(base, CH0 * C)], ia1.at[pl.ds(0, CH0 * C)])
        pltpu.sync_copy(i2_hbm.at[pl.ds(base, CH0 * C)], ia2.at[pl.ds(0, CH0 * C)])

    @pl.when(cid == 1)
    def _():
        pltpu.sync_copy(i0_hbm.at[pl.ds(base, CH1 * C)], ia0.at[pl.ds(0, CH1 * C)])
        pltpu.sync_copy(i1_hbm.at[pl.ds(base, CH1 * C)], ia1.at[pl.ds(0, CH1 * C)])
        pltpu.sync_copy(i2_hbm.at[pl.ds(base, CH1 * C)], ia2.at[pl.ds(0, CH1 * C)])

    banks = ((r0a, r1a, r2a, oa, sem_a, osem_a),
             (r0b, r1b, r2b, ob, sem_b, osem_b))

    def gather_cps(c, bk):
        r0, r1, r2, _, sem, _ = banks[bk]
        off = c * C
        return (pltpu.make_async_copy(p0_hbm.at[ia0.at[pl.ds(off, C)]], r0, sem),
                pltpu.make_async_copy(p1_hbm.at[ia1.at[pl.ds(off, C)]], r1, sem),
                pltpu.make_async_copy(p2_hbm.at[ia2.at[pl.ds(off, C)]], r2, sem))

    def gather_start(c, bk):
        for cp in gather_cps(c, bk):
            cp.start()

    def gather_wait(c, bk):
        for cp in gather_cps(c, bk):
            cp.wait()

    def out_ops(c, bk, start):
        _, _, _, o, _, osem = banks[bk]
        og = base + c * C

        @pl.when(og <= FULL_MAX)
        def _():
            cp = pltpu.make_async_copy(o, out_hbm.at[pl.ds(og, C)], osem)
            cp.start() if start else cp.wait()

        @pl.when(og == STRAD_OFF)
        def _():
            cp = pltpu.make_async_copy(o.at[pl.ds(0, STRAD_ROWS)],
                                       out_hbm.at[pl.ds(og, STRAD_ROWS)], osem)
            cp.start() if start else cp.wait()

    himask = jnp.int32(-65536)

    def compute(bk):
        r0, r1, r2, o, _, _ = banks[bk]

        def row(rr, carry):
            v0 = r0[rr, :]
            v1 = r1[rr, :]
            v2 = r2[rr, :]
            lo = (lax.bitcast_convert_type(v0 << 16, jnp.float32)
                  + lax.bitcast_convert_type(v1 << 16, jnp.float32)
                  + lax.bitcast_convert_type(v2 << 16, jnp.float32))
            hi = (lax.bitcast_convert_type(v0 & himask, jnp.float32)
                  + lax.bitcast_convert_type(v1 & himask, jnp.float32)
                  + lax.bitcast_convert_type(v2 & himask, jnp.float32))
            o[rr, pl.ds(0, DW)] = jnp.where(lo > 0.0, lo, jnp.exp(lo) - 1.0)
            o[rr, pl.ds(DW, DW)] = jnp.where(hi > 0.0, hi, jnp.exp(hi) - 1.0)
            return carry

        lax.fori_loop(0, C, row, 0)

    gather_start(0, 0)
    gather_start(1, 1)

    def pair(i, carry):
        c0 = 2 * i
        c1 = 2 * i + 1

        gather_wait(c0, 0)

        @pl.when(c0 >= 2)
        def _():
            out_ops(c0 - 2, 0, start=False)

        compute(0)
        out_ops(c0, 0, start=True)

        @pl.when(c0 + 2 < my_chunks)
        def _():
            gather_start(c0 + 2, 0)

        gather_wait(c1, 1)

        @pl.when(c1 >= 3)
        def _():
            out_ops(c1 - 2, 1, start=False)

        compute(1)
        out_ops(c1, 1, start=True)

        @pl.when(c1 + 2 < my_chunks)
        def _():
            gather_start(c1 + 2, 1)

        return carry

    lax.fori_loop(0, my_chunks // 2, pair, 0)
    out_ops(my_chunks - 2, 0, start=False)
    out_ops(my_chunks - 1, 1, start=False)


def _gather_combine(p0, p1, p2, i0, i1, i2):
    mesh = plsc.VectorSubcoreMesh(core_axis_name="c", subcore_axis_name="s")
    fn = functools.partial(
        pl.kernel,
        out_type=jax.ShapeDtypeStruct((N_ACTS, D), jnp.float32),
        mesh=mesh,
        scratch_types=[
            pltpu.VMEM((MAXCH * C,), jnp.int32),
            pltpu.VMEM((MAXCH * C,), jnp.int32),
            pltpu.VMEM((MAXCH * C,), jnp.int32),
            pltpu.VMEM((C, DW), jnp.int32),
            pltpu.VMEM((C, DW), jnp.int32),
            pltpu.VMEM((C, DW), jnp.int32),
            pltpu.VMEM((C, DW), jnp.int32),
            pltpu.VMEM((C, DW), jnp.int32),
            pltpu.VMEM((C, DW), jnp.int32),
            pltpu.VMEM((C, D), jnp.float32),
            pltpu.VMEM((C, D), jnp.float32),
            pltpu.SemaphoreType.DMA,
            pltpu.SemaphoreType.DMA,
            pltpu.SemaphoreType.DMA,
            pltpu.SemaphoreType.DMA,
        ],
    )(_sc_body)
    return fn(p0, p1, p2, i0, i1, i2)


def kernel(prop_feats, idx0, idx1, idx2, W, b):
    prop2d = prop_feats.reshape(N_PROPS, D)
    p0, p1, p2 = _project_tables(prop2d, W, b)
    pad = jnp.zeros((PAD_ACTS - N_ACTS,), jnp.int32)
    i0 = jnp.concatenate([idx0, pad])
    i1 = jnp.concatenate([idx1, pad])
    i2 = jnp.concatenate([idx2, pad])
    out = _gather_combine(p0, p1, p2, i0, i1, i2)
    return out.reshape(1, N_ACTS, D)


# speedup vs baseline: 1.0587x; 1.0587x over previous
import functools

import jax
import jax.numpy as jnp
from jax import lax
from jax.experimental import pallas as pl
from jax.experimental.pallas import tpu as pltpu
from jax.experimental.pallas import tpu_sc as plsc

N_PROPS = 50000
N_ACTS = 100000
D = 256
DW = D // 2
NC, NS = 2, 16
NW = NC * NS
C = 64
CH0 = 70
CH1 = 30
SLAB = (CH0 + CH1) * C
MAXCH = max(CH0, CH1)
PAD_ACTS = NS * SLAB

FULL_MAX = N_ACTS - C
STRAD_OFF = (N_ACTS // C) * C
STRAD_ROWS = N_ACTS - STRAD_OFF

MM_ROWS = 2000


def _mm_body(x_ref, w_ref, b_ref, o0_ref, o1_ref, o2_ref):
    x = x_ref[...]
    for k, o_ref in enumerate((o0_ref, o1_ref, o2_ref)):
        p = jnp.dot(x, w_ref[k * D:(k + 1) * D, :],
                    preferred_element_type=jnp.float32)
        if k == 0:
            p = p + b_ref[...]
        bits = lax.bitcast_convert_type(p, jnp.uint32)
        r16 = (bits + 0x7FFF + ((bits >> 16) & 1)) >> 16
        w = r16[:, :DW] | (r16[:, DW:] << 16)
        o_ref[...] = lax.bitcast_convert_type(w, jnp.int32)


def _project_tables(prop2d, W, b):
    return pl.pallas_call(
        _mm_body,
        grid=(N_PROPS // MM_ROWS,),
        in_specs=[
            pl.BlockSpec((MM_ROWS, D), lambda i: (i, 0)),
            pl.BlockSpec((3 * D, D), lambda i: (0, 0)),
            pl.BlockSpec((1, D), lambda i: (0, 0)),
        ],
        out_specs=[pl.BlockSpec((MM_ROWS, DW), lambda i: (i, 0))] * 3,
        out_shape=[jax.ShapeDtypeStruct((N_PROPS, DW), jnp.int32)] * 3,
    )(prop2d, W, b.reshape(1, D))


def _sc_body(p0_hbm, p1_hbm, p2_hbm, i0_hbm, i1_hbm, i2_hbm, out_hbm,
             ia0, ia1, ia2, r0a, r1a, r2a, r0b, r1b, r2b, oa, ob,
             sem_a, sem_b, osem_a, osem_b):
    cid = lax.axis_index("c")
    sid = lax.axis_index("s")
    my_chunks = jnp.where(cid == 0, CH0, CH1)
    base = sid * SLAB + cid * (CH0 * C)

    @pl.when(cid == 0)
    def _():
        pltpu.sync_copy(i0_hbm.at[pl.ds(base, CH0 * C)], ia0.at[pl.ds(0, CH0 * C)])
        pltpu.sync_copy(i1_hbm.at[pl.ds(base, CH0 * C)], ia1.at[pl.ds(0, CH0 * C)])
        pltpu.sync_copy(i2_hbm.at[pl.ds(base, CH0 * C)], ia2.at[pl.ds(0, CH0 * C)])

    @pl.when(cid == 1)
    def _():
        pltpu.sync_copy(i0_hbm.at[pl.ds(base, CH1 * C)], ia0.at[pl.ds(0, CH1 * C)])
        pltpu.sync_copy(i1_hbm.at[pl.ds(base, CH1 * C)], ia1.at[pl.ds(0, CH1 * C)])
        pltpu.sync_copy(i2_hbm.at[pl.ds(base, CH1 * C)], ia2.at[pl.ds(0, CH1 * C)])

    banks = ((r0a, r1a, r2a, oa, sem_a, osem_a),
             (r0b, r1b, r2b, ob, sem_b, osem_b))

    def gather_cps(c, bk):
        r0, r1, r2, _, sem, _ = banks[bk]
        off = c * C
        return (pltpu.make_async_copy(p0_hbm.at[ia0.at[pl.ds(off, C)]], r0, sem),
                pltpu.make_async_copy(p1_hbm.at[ia1.at[pl.ds(off, C)]], r1, sem),
                pltpu.make_async_copy(p2_hbm.at[ia2.at[pl.ds(off, C)]], r2, sem))

    def gather_start(c, bk):
        for cp in gather_cps(c, bk):
            cp.start()

    def gather_wait(c, bk):
        for cp in gather_cps(c, bk):
            cp.wait()

    def out_ops(c, bk, start):
        _, _, _, o, _, osem = banks[bk]
        og = base + c * C

        @pl.when(og <= FULL_MAX)
        def _():
            cp = pltpu.make_async_copy(o, out_hbm.at[pl.ds(og, C)], osem)
            cp.start() if start else cp.wait()

        @pl.when(og == STRAD_OFF)
        def _():
            cp = pltpu.make_async_copy(o.at[pl.ds(0, STRAD_ROWS)],
                                       out_hbm.at[pl.ds(og, STRAD_ROWS)], osem)
            cp.start() if start else cp.wait()

    himask = jnp.int32(-65536)

    def compute(bk):
        r0, r1, r2, o, _, _ = banks[bk]

        def row(rr, carry):
            v0 = r0[rr, :]
            v1 = r1[rr, :]
            v2 = r2[rr, :]
            lo = (lax.bitcast_convert_type(v0 << 16, jnp.float32)
                  + lax.bitcast_convert_type(v1 << 16, jnp.float32)
                  + lax.bitcast_convert_type(v2 << 16, jnp.float32))
            hi = (lax.bitcast_convert_type(v0 & himask, jnp.float32)
                  + lax.bitcast_convert_type(v1 & himask, jnp.float32)
                  + lax.bitcast_convert_type(v2 & himask, jnp.float32))
            o[rr, pl.ds(0, DW)] = jnp.where(lo > 0.0, lo, jnp.exp(lo) - 1.0)
            o[rr, pl.ds(DW, DW)] = jnp.where(hi > 0.0, hi, jnp.exp(hi) - 1.0)
            return carry

        lax.fori_loop(0, C, row, 0)

    gather_start(0, 0)
    gather_start(1, 1)

    def pair(i, carry):
        c0 = 2 * i
        c1 = 2 * i + 1

        gather_wait(c0, 0)

        @pl.when(c0 >= 2)
        def _():
            out_ops(c0 - 2, 0, start=False)

        compute(0)
        out_ops(c0, 0, start=True)

        @pl.when(c0 + 2 < my_chunks)
        def _():
            gather_start(c0 + 2, 0)

        gather_wait(c1, 1)

        @pl.when(c1 >= 3)
        def _():
            out_ops(c1 - 2, 1, start=False)

        compute(1)
        out_ops(c1, 1, start=True)

        @pl.when(c1 + 2 < my_chunks)
        def _():
            gather_start(c1 + 2, 1)

        return carry

    lax.fori_loop(0, my_chunks // 2, pair, 0)
    out_ops(my_chunks - 2, 0, start=False)
    out_ops(my_chunks - 1, 1, start=False)


def _gather_combine(p0, p1, p2, i0, i1, i2):
    mesh = plsc.VectorSubcoreMesh(core_axis_name="c", subcore_axis_name="s")
    fn = functools.partial(
        pl.kernel,
        out_type=jax.ShapeDtypeStruct((N_ACTS, D), jnp.float32),
        mesh=mesh,
        scratch_types=[
            pltpu.VMEM((MAXCH * C,), jnp.int32),
            pltpu.VMEM((MAXCH * C,), jnp.int32),
            pltpu.VMEM((MAXCH * C,), jnp.int32),
            pltpu.VMEM((C, DW), jnp.int32),
            pltpu.VMEM((C, DW), jnp.int32),
            pltpu.VMEM((C, DW), jnp.int32),
            pltpu.VMEM((C, DW), jnp.int32),
            pltpu.VMEM((C, DW), jnp.int32),
            pltpu.VMEM((C, DW), jnp.int32),
            pltpu.VMEM((C, D), jnp.float32),
            pltpu.VMEM((C, D), jnp.float32),
            pltpu.SemaphoreType.DMA,
            pltpu.SemaphoreType.DMA,
            pltpu.SemaphoreType.DMA,
            pltpu.SemaphoreType.DMA,
        ],
    )(_sc_body)
    return fn(p0, p1, p2, i0, i1, i2)


def kernel(prop_feats, idx0, idx1, idx2, W, b):
    prop2d = prop_feats.reshape(N_PROPS, D)
    p0, p1, p2 = _project_tables(prop2d, W, b)
    pad = jnp.zeros((PAD_ACTS - N_ACTS,), jnp.int32)
    i0 = jnp.concatenate([idx0, pad])
    i1 = jnp.concatenate([idx1, pad])
    i2 = jnp.concatenate([idx2, pad])
    out = _gather_combine(p0, p1, p2, i0, i1, i2)
    return out.reshape(1, N_ACTS, D)


# speedup vs baseline: 1.0951x; 1.0344x over previous
import functools

import jax
import jax.numpy as jnp
from jax import lax
from jax.experimental import pallas as pl
from jax.experimental.pallas import tpu as pltpu
from jax.experimental.pallas import tpu_sc as plsc

N_PROPS = 50000
N_ACTS = 100000
D = 256
DW = D // 2
NC, NS = 2, 16
NW = NC * NS
C = 64
CH0 = 70
CH1 = 30
SLAB = (CH0 + CH1) * C
MAXCH = max(CH0, CH1)
PAD_ACTS = NS * SLAB

FULL_MAX = N_ACTS - C
STRAD_OFF = (N_ACTS // C) * C
STRAD_ROWS = N_ACTS - STRAD_OFF

MM_ROWS = 5000


def _mm_body(x_ref, w_ref, b_ref, o0_ref, o1_ref, o2_ref):
    x = x_ref[...]
    for k, o_ref in enumerate((o0_ref, o1_ref, o2_ref)):
        p = jnp.dot(x, w_ref[k * D:(k + 1) * D, :],
                    preferred_element_type=jnp.float32)
        if k == 0:
            p = p + b_ref[...]
        bits = lax.bitcast_convert_type(p, jnp.uint32)
        r16 = (bits + 0x7FFF + ((bits >> 16) & 1)) >> 16
        w = r16[:, :DW] | (r16[:, DW:] << 16)
        o_ref[...] = lax.bitcast_convert_type(w, jnp.int32)


def _project_tables(prop2d, W, b):
    return pl.pallas_call(
        _mm_body,
        grid=(N_PROPS // MM_ROWS,),
        in_specs=[
            pl.BlockSpec((MM_ROWS, D), lambda i: (i, 0)),
            pl.BlockSpec((3 * D, D), lambda i: (0, 0)),
            pl.BlockSpec((1, D), lambda i: (0, 0)),
        ],
        out_specs=[pl.BlockSpec((MM_ROWS, DW), lambda i: (i, 0))] * 3,
        out_shape=[jax.ShapeDtypeStruct((N_PROPS, DW), jnp.int32)] * 3,
    )(prop2d, W, b.reshape(1, D))


def _sc_body(p0_hbm, p1_hbm, p2_hbm, i0_hbm, i1_hbm, i2_hbm, out_hbm,
             ia0, ia1, ia2, r0a, r1a, r2a, r0b, r1b, r2b, oa, ob,
             sem_a, sem_b, osem_a, osem_b):
    cid = lax.axis_index("c")
    sid = lax.axis_index("s")
    my_chunks = jnp.where(cid == 0, CH0, CH1)
    base = sid * SLAB + cid * (CH0 * C)

    @pl.when(cid == 0)
    def _():
        pltpu.sync_copy(i0_hbm.at[pl.ds(base, CH0 * C)], ia0.at[pl.ds(0, CH0 * C)])
        pltpu.sync_copy(i1_hbm.at[pl.ds(base, CH0 * C)], ia1.at[pl.ds(0, CH0 * C)])
        pltpu.sync_copy(i2_hbm.at[pl.ds(base, CH0 * C)], ia2.at[pl.ds(0, CH0 * C)])

    @pl.when(cid == 1)
    def _():
        pltpu.sync_copy(i0_hbm.at[pl.ds(base, CH1 * C)], ia0.at[pl.ds(0, CH1 * C)])
        pltpu.sync_copy(i1_hbm.at[pl.ds(base, CH1 * C)], ia1.at[pl.ds(0, CH1 * C)])
        pltpu.sync_copy(i2_hbm.at[pl.ds(base, CH1 * C)], ia2.at[pl.ds(0, CH1 * C)])

    banks = ((r0a, r1a, r2a, oa, sem_a, osem_a),
             (r0b, r1b, r2b, ob, sem_b, osem_b))

    def gather_cps(c, bk):
        r0, r1, r2, _, sem, _ = banks[bk]
        off = c * C
        return (pltpu.make_async_copy(p0_hbm.at[ia0.at[pl.ds(off, C)]], r0, sem),
                pltpu.make_async_copy(p1_hbm.at[ia1.at[pl.ds(off, C)]], r1, sem),
                pltpu.make_async_copy(p2_hbm.at[ia2.at[pl.ds(off, C)]], r2, sem))

    def gather_start(c, bk):
        for cp in gather_cps(c, bk):
            cp.start()

    def gather_wait(c, bk):
        for cp in gather_cps(c, bk):
            cp.wait()

    def out_ops(c, bk, start):
        _, _, _, o, _, osem = banks[bk]
        og = base + c * C

        @pl.when(og <= FULL_MAX)
        def _():
            cp = pltpu.make_async_copy(o, out_hbm.at[pl.ds(og, C)], osem)
            cp.start() if start else cp.wait()

        @pl.when(og == STRAD_OFF)
        def _():
            cp = pltpu.make_async_copy(o.at[pl.ds(0, STRAD_ROWS)],
                                       out_hbm.at[pl.ds(og, STRAD_ROWS)], osem)
            cp.start() if start else cp.wait()

    himask = jnp.int32(-65536)

    def compute(bk):
        r0, r1, r2, o, _, _ = banks[bk]

        def row(rr, carry):
            v0 = r0[rr, :]
            v1 = r1[rr, :]
            v2 = r2[rr, :]
            lo = (lax.bitcast_convert_type(v0 << 16, jnp.float32)
                  + lax.bitcast_convert_type(v1 << 16, jnp.float32)
                  + lax.bitcast_convert_type(v2 << 16, jnp.float32))
            hi = (lax.bitcast_convert_type(v0 & himask, jnp.float32)
                  + lax.bitcast_convert_type(v1 & himask, jnp.float32)
                  + lax.bitcast_convert_type(v2 & himask, jnp.float32))
            o[rr, pl.ds(0, DW)] = jnp.where(lo > 0.0, lo, jnp.exp(lo) - 1.0)
            o[rr, pl.ds(DW, DW)] = jnp.where(hi > 0.0, hi, jnp.exp(hi) - 1.0)
            return carry

        lax.fori_loop(0, C, row, 0)

    gather_start(0, 0)
    gather_start(1, 1)

    def pair(i, carry):
        c0 = 2 * i
        c1 = 2 * i + 1

        gather_wait(c0, 0)

        @pl.when(c0 >= 2)
        def _():
            out_ops(c0 - 2, 0, start=False)

        compute(0)
        out_ops(c0, 0, start=True)

        @pl.when(c0 + 2 < my_chunks)
        def _():
            gather_start(c0 + 2, 0)

        gather_wait(c1, 1)

        @pl.when(c1 >= 3)
        def _():
            out_ops(c1 - 2, 1, start=False)

        compute(1)
        out_ops(c1, 1, start=True)

        @pl.when(c1 + 2 < my_chunks)
        def _():
            gather_start(c1 + 2, 1)

        return carry

    lax.fori_loop(0, my_chunks // 2, pair, 0)
    out_ops(my_chunks - 2, 0, start=False)
    out_ops(my_chunks - 1, 1, start=False)


def _gather_combine(p0, p1, p2, i0, i1, i2):
    mesh = plsc.VectorSubcoreMesh(core_axis_name="c", subcore_axis_name="s")
    fn = functools.partial(
        pl.kernel,
        out_type=jax.ShapeDtypeStruct((N_ACTS, D), jnp.float32),
        mesh=mesh,
        scratch_types=[
            pltpu.VMEM((MAXCH * C,), jnp.int32),
            pltpu.VMEM((MAXCH * C,), jnp.int32),
            pltpu.VMEM((MAXCH * C,), jnp.int32),
            pltpu.VMEM((C, DW), jnp.int32),
            pltpu.VMEM((C, DW), jnp.int32),
            pltpu.VMEM((C, DW), jnp.int32),
            pltpu.VMEM((C, DW), jnp.int32),
            pltpu.VMEM((C, DW), jnp.int32),
            pltpu.VMEM((C, DW), jnp.int32),
            pltpu.VMEM((C, D), jnp.float32),
            pltpu.VMEM((C, D), jnp.float32),
            pltpu.SemaphoreType.DMA,
            pltpu.SemaphoreType.DMA,
            pltpu.SemaphoreType.DMA,
            pltpu.SemaphoreType.DMA,
        ],
    )(_sc_body)
    return fn(p0, p1, p2, i0, i1, i2)


def kernel(prop_feats, idx0, idx1, idx2, W, b):
    prop2d = prop_feats.reshape(N_PROPS, D)
    p0, p1, p2 = _project_tables(prop2d, W, b)
    pad = jnp.zeros((PAD_ACTS - N_ACTS,), jnp.int32)
    i0 = jnp.concatenate([idx0, pad])
    i1 = jnp.concatenate([idx1, pad])
    i2 = jnp.concatenate([idx2, pad])
    out = _gather_combine(p0, p1, p2, i0, i1, i2)
    return out.reshape(1, N_ACTS, D)
